# baseline (speedup 1.0000x reference)
import functools

import numpy as np

import jax
import jax.numpy as jnp
from jax import lax
from jax.experimental import pallas as pl
from jax.experimental.pallas import tpu as pltpu
from jax.experimental.pallas import tpu_sc as plsc


def _make_sc_gather(V, M, B, D):
    info = plsc.get_sparse_core_info()
    NC, NS = info.num_cores, info.num_subcores
    NW = NC * NS
    BH = B // 2
    P = M * BH
    p_per_w = P // NW
    CHP = 64
    n_ch = p_per_w // CHP
    assert p_per_w % CHP == 0 and BH % CHP == 0
    mesh = plsc.VectorSubcoreMesh(core_axis_name="c", subcore_axis_name="s")

    @functools.partial(
        pl.kernel,
        mesh=mesh,
        out_type=jax.ShapeDtypeStruct((M, BH, 2, D), jnp.float32),
        compiler_params=pltpu.CompilerParams(use_tc_tiling_on_sc=False),
        scratch_types=[
            pltpu.VMEM((2 * p_per_w,), jnp.int32),
            pltpu.VMEM((2, 2, CHP, D), jnp.float32),
            pltpu.SemaphoreType.DMA,
        ],
    )
    def gather_k(idx_hbm, table_hbm, out_hbm, idx_v, rows_v, sem):
        wid = lax.axis_index("s") * NC + lax.axis_index("c")
        base = wid * 2 * p_per_w
        pltpu.sync_copy(idx_hbm.at[pl.ds(base, 2 * p_per_w)], idx_v)

        def start(c):
            buf = rows_v.at[lax.rem(c, 2)]
            pltpu.async_copy(
                table_hbm.at[idx_v.at[pl.ds(c * 2 * CHP, CHP)]],
                buf.at[0],
                sem,
            )
            pltpu.async_copy(
                table_hbm.at[idx_v.at[pl.ds(c * 2 * CHP + CHP, CHP)]],
                buf.at[1],
                sem,
            )

        def drain_flush(c):
            buf = rows_v.at[lax.rem(c, 2)]
            pltpu.make_async_copy(
                table_hbm.at[idx_v.at[pl.ds(c * 2 * CHP, CHP)]], buf.at[0], sem
            ).wait()
            pltpu.make_async_copy(
                table_hbm.at[idx_v.at[pl.ds(c * 2 * CHP, CHP)]], buf.at[1], sem
            ).wait()
            p0 = wid * p_per_w + c * CHP
            m_c = lax.div(p0, BH)
            j0 = lax.rem(p0, BH)
            pltpu.sync_copy(buf.at[0], out_hbm.at[m_c, pl.ds(j0, CHP), 0])
            pltpu.sync_copy(buf.at[1], out_hbm.at[m_c, pl.ds(j0, CHP), 1])

        start(0)

        def body(c, carry):
            @pl.when(c + 1 < n_ch)
            def _():
                start(c + 1)

            drain_flush(c)
            return carry

        lax.fori_loop(0, n_ch, body, 0)

    return gather_k


def _attn_body(m_ref, wbd_ref, b1b_ref, w2bd_ref, o_ref, s_ref):
    M = m_ref.shape[0]
    D = o_ref.shape[1] // 2
    wbd = wbd_ref[...]
    b1b = b1b_ref[...]
    w2bd = w2bd_ref[...]
    for m in range(M):
        h = jnp.maximum(
            jnp.dot(m_ref[m], wbd, preferred_element_type=jnp.float32) + b1b, 0.0
        )
        s2 = jnp.dot(h, w2bd, preferred_element_type=jnp.float32)
        s_ref[:, m : m + 1] = s2[:, 0:1]
        s_ref[:, M + m : M + m + 1] = s2[:, 1:2]
    sc = s_ref[...]
    sc0 = sc[:, :M]
    sc1 = sc[:, M:]
    e0 = jnp.exp(sc0 - jnp.max(sc0, axis=1, keepdims=True))
    att0 = e0 / jnp.sum(e0, axis=1, keepdims=True)
    e1 = jnp.exp(sc1 - jnp.max(sc1, axis=1, keepdims=True))
    att1 = e1 / jnp.sum(e1, axis=1, keepdims=True)
    G = m_ref.shape[1]
    lane = lax.broadcasted_iota(jnp.int32, (G, 2 * D), 1)
    low = lane < D
    acc = jnp.zeros((G, 2 * D), jnp.float32)
    for m in range(M):
        a2 = jnp.where(low, att0[:, m : m + 1], att1[:, m : m + 1])
        acc = acc + a2 * m_ref[m]
    o_ref[...] = acc


def kernel(nodes, to_neighs, u2e_weight, ge_w1, ge_b1, ge_w2, ge_b2):
    B, M = nodes.shape
    V, D = u2e_weight.shape
    H = ge_w1.shape[1]

    BH = B // 2
    p = np.arange(M * BH)
    m_of_p, j_of_p = p // BH, p % BH
    low_j = (m_of_p * B + j_of_p).reshape(-1, 64)
    high_j = (m_of_p * B + BH + j_of_p).reshape(-1, 64)
    perm = jnp.asarray(
        np.concatenate([low_j, high_j], axis=1).reshape(-1), dtype=jnp.int32
    )
    idx = jnp.take(nodes.T.reshape(-1).astype(jnp.int32), perm)
    staged4 = _make_sc_gather(V, M, B, D)(idx, u2e_weight)
    staged = staged4.reshape(M, BH, 2 * D)

    z = jnp.zeros_like(ge_w1)
    wbd = jnp.concatenate(
        [jnp.concatenate([ge_w1, z], axis=1), jnp.concatenate([z, ge_w1], axis=1)],
        axis=0,
    )
    b1b = jnp.concatenate([ge_b1, ge_b1]).reshape(1, 2 * H)
    w2c = ge_w2[:, 0]
    w2bd = jnp.zeros((2 * H, 2), jnp.float32)
    w2bd = w2bd.at[:H, 0].set(w2c).at[H:, 1].set(w2c)

    Gp = 512
    grid = (B // 2 // Gp,)
    out2 = pl.pallas_call(
        _attn_body,
        grid=grid,
        in_specs=[
            pl.BlockSpec((M, Gp, 2 * D), lambda i: (0, i, 0)),
            pl.BlockSpec((2 * D, 2 * H), lambda i: (0, 0)),
            pl.BlockSpec((1, 2 * H), lambda i: (0, 0)),
            pl.BlockSpec((2 * H, 2), lambda i: (0, 0)),
        ],
        out_specs=pl.BlockSpec((Gp, 2 * D), lambda i: (i, 0)),
        out_shape=jax.ShapeDtypeStruct((B // 2, 2 * D), jnp.float32),
        scratch_shapes=[pltpu.VMEM((Gp, 2 * M), jnp.float32)],
    )(staged, wbd, b1b, w2bd)
    return jnp.concatenate([out2[:, :D], out2[:, D:]], axis=0)


# speedup vs baseline: 2.9940x; 2.9940x over previous
import functools

import numpy as np

import jax
import jax.numpy as jnp
from jax import lax
from jax.experimental import pallas as pl
from jax.experimental.pallas import tpu as pltpu
from jax.experimental.pallas import tpu_sc as plsc


def _make_sc_gather(V, M, B, D):
    info = plsc.get_sparse_core_info()
    NC, NS = info.num_cores, info.num_subcores
    NW = NC * NS
    BH = B // 2
    P = M * BH
    p_per_w = P // NW
    CHP = 64
    n_ch = p_per_w // CHP
    assert p_per_w % CHP == 0 and BH % CHP == 0
    mesh = plsc.VectorSubcoreMesh(core_axis_name="c", subcore_axis_name="s")

    @functools.partial(
        pl.kernel,
        mesh=mesh,
        out_type=jax.ShapeDtypeStruct((M, BH, 2 * D), jnp.float32),
        compiler_params=pltpu.CompilerParams(use_tc_tiling_on_sc=False),
        scratch_types=[
            pltpu.VMEM((2 * p_per_w,), jnp.int32),
            pltpu.VMEM((2, 2, CHP, D), jnp.float32),
            pltpu.SemaphoreType.DMA,
        ],
    )
    def gather_k(idx_hbm, table_hbm, out_hbm, idx_v, rows_v, sem):
        wid = lax.axis_index("s") * NC + lax.axis_index("c")
        base = wid * 2 * p_per_w
        pltpu.sync_copy(idx_hbm.at[pl.ds(base, 2 * p_per_w)], idx_v)

        def start(c):
            buf = rows_v.at[lax.rem(c, 2)]
            pltpu.async_copy(
                table_hbm.at[idx_v.at[pl.ds(c * 2 * CHP, CHP)]],
                buf.at[0],
                sem,
            )
            pltpu.async_copy(
                table_hbm.at[idx_v.at[pl.ds(c * 2 * CHP + CHP, CHP)]],
                buf.at[1],
                sem,
            )

        def drain_flush(c):
            buf = rows_v.at[lax.rem(c, 2)]
            pltpu.make_async_copy(
                table_hbm.at[idx_v.at[pl.ds(c * 2 * CHP, CHP)]], buf.at[0], sem
            ).wait()
            pltpu.make_async_copy(
                table_hbm.at[idx_v.at[pl.ds(c * 2 * CHP, CHP)]], buf.at[1], sem
            ).wait()
            p0 = wid * p_per_w + c * CHP
            m_c = lax.div(p0, BH)
            j0 = lax.rem(p0, BH)
            pltpu.sync_copy(buf.at[0], out_hbm.at[m_c, pl.ds(j0, CHP), pl.ds(0, D)])
            pltpu.sync_copy(buf.at[1], out_hbm.at[m_c, pl.ds(j0, CHP), pl.ds(D, D)])

        start(0)

        def body(c, carry):
            @pl.when(c + 1 < n_ch)
            def _():
                start(c + 1)

            drain_flush(c)
            return carry

        lax.fori_loop(0, n_ch, body, 0)

    return gather_k


def _attn_body(m_ref, wbd_ref, b1b_ref, w2bd_ref, o_ref, s_ref):
    M = m_ref.shape[0]
    D = o_ref.shape[1] // 2
    wbd = wbd_ref[...]
    b1b = b1b_ref[...]
    w2bd = w2bd_ref[...]
    for m in range(M):
        h = jnp.maximum(
            jnp.dot(m_ref[m], wbd, preferred_element_type=jnp.float32) + b1b, 0.0
        )
        s2 = jnp.dot(h, w2bd, preferred_element_type=jnp.float32)
        s_ref[:, m : m + 1] = s2[:, 0:1]
        s_ref[:, M + m : M + m + 1] = s2[:, 1:2]
    sc = s_ref[...]
    sc0 = sc[:, :M]
    sc1 = sc[:, M:]
    e0 = jnp.exp(sc0 - jnp.max(sc0, axis=1, keepdims=True))
    att0 = e0 / jnp.sum(e0, axis=1, keepdims=True)
    e1 = jnp.exp(sc1 - jnp.max(sc1, axis=1, keepdims=True))
    att1 = e1 / jnp.sum(e1, axis=1, keepdims=True)
    G = m_ref.shape[1]
    lane = lax.broadcasted_iota(jnp.int32, (G, 2 * D), 1)
    low = lane < D
    acc = jnp.zeros((G, 2 * D), jnp.float32)
    for m in range(M):
        a2 = jnp.where(low, att0[:, m : m + 1], att1[:, m : m + 1])
        acc = acc + a2 * m_ref[m]
    o_ref[...] = acc


def kernel(nodes, to_neighs, u2e_weight, ge_w1, ge_b1, ge_w2, ge_b2):
    B, M = nodes.shape
    V, D = u2e_weight.shape
    H = ge_w1.shape[1]

    BH = B // 2
    nt = nodes.T.astype(jnp.int32)
    lo = nt[:, :BH].reshape(M, BH // 64, 1, 64)
    hi = nt[:, BH:].reshape(M, BH // 64, 1, 64)
    idx = jnp.concatenate([lo, hi], axis=2).reshape(M * B)
    staged = _make_sc_gather(V, M, B, D)(idx, u2e_weight)

    z = jnp.zeros_like(ge_w1)
    wbd = jnp.concatenate(
        [jnp.concatenate([ge_w1, z], axis=1), jnp.concatenate([z, ge_w1], axis=1)],
        axis=0,
    )
    b1b = jnp.concatenate([ge_b1, ge_b1]).reshape(1, 2 * H)
    w2c = ge_w2[:, 0]
    w2bd = jnp.zeros((2 * H, 2), jnp.float32)
    w2bd = w2bd.at[:H, 0].set(w2c).at[H:, 1].set(w2c)

    Gp = 512
    grid = (B // 2 // Gp,)
    out2 = pl.pallas_call(
        _attn_body,
        grid=grid,
        in_specs=[
            pl.BlockSpec((M, Gp, 2 * D), lambda i: (0, i, 0)),
            pl.BlockSpec((2 * D, 2 * H), lambda i: (0, 0)),
            pl.BlockSpec((1, 2 * H), lambda i: (0, 0)),
            pl.BlockSpec((2 * H, 2), lambda i: (0, 0)),
        ],
        out_specs=pl.BlockSpec((Gp, 2 * D), lambda i: (i, 0)),
        out_shape=jax.ShapeDtypeStruct((B // 2, 2 * D), jnp.float32),
        scratch_shapes=[pltpu.VMEM((Gp, 2 * M), jnp.float32)],
    )(staged, wbd, b1b, w2bd)
    return jnp.concatenate([out2[:, :D], out2[:, D:]], axis=0)
